# speedup vs baseline: 1.3004x; 1.3004x over previous
import functools

import jax
import jax.numpy as jnp
from jax import lax
from jax.experimental import pallas as pl
from jax.experimental.pallas import tpu as pltpu
from jax.experimental.pallas import tpu_sc as plsc

_NC = 2
_NS = 16
_NW = _NC * _NS
_C = 128
_NB = 4


def _make_body(nch, nblk, d):
    def body(idx_hbm, table_hbm, out_hbm,
             idx_v, r0, r1, r2, r3, g0, g1, g2, g3, s0, s1, s2, s3):
        rows = (r0, r1, r2, r3)
        gsem = (g0, g1, g2, g3)
        ssem = (s0, s1, s2, s3)

        wid = lax.axis_index("s") * _NC + lax.axis_index("c")
        row0 = wid * nch

        pltpu.sync_copy(idx_hbm.at[pl.ds(row0, nch)], idx_v)

        def gather(c, b):
            pltpu.make_async_copy(table_hbm.at[idx_v.at[c]], rows[b], gsem[b]).start()

        def wait_gather(b):
            pltpu.make_async_copy(table_hbm.at[pl.ds(0, _C)], rows[b], gsem[b]).wait()

        def store(c, b):
            pltpu.make_async_copy(
                rows[b], out_hbm.at[pl.ds((row0 + c) * _C, _C)], ssem[b]).start()

        def wait_store(b):
            pltpu.make_async_copy(rows[b], out_hbm.at[pl.ds(0, _C)], ssem[b]).wait()

        for c in range(_NB - 1):
            gather(c, c)

        for b in range(_NB):
            pb = (b + _NB - 1) % _NB
            if b >= 1:
                wait_store(pb)
            gather(b + _NB - 1, pb)
            wait_gather(b)
            store(b, b)

        def blk(i, carry):
            c0 = i * _NB
            for b in range(_NB):
                c = c0 + b
                pb = (b + _NB - 1) % _NB
                wait_store(pb)
                gather(c + _NB - 1, pb)
                wait_gather(b)
                store(c, b)
            return carry
        lax.fori_loop(1, nblk - 1, blk, 0)

        c0 = (nblk - 1) * _NB
        wait_store(_NB - 1)
        gather(c0 + _NB - 1, _NB - 1)
        wait_gather(0)
        store(c0, 0)
        for b in range(1, _NB):
            wait_gather(b)
            store(c0 + b, b)
        for b in range(_NB):
            wait_store(b)

    return body


@functools.lru_cache(maxsize=None)
def _make_call(total, d):
    assert total % (_NW * _C) == 0
    nch = total // (_NW * _C)
    assert nch % _NB == 0 and nch // _NB >= 2
    nblk = nch // _NB

    return pl.kernel(
        _make_body(nch, nblk, d),
        out_type=jax.ShapeDtypeStruct((total, d), jnp.float32),
        mesh=plsc.VectorSubcoreMesh(core_axis_name="c", subcore_axis_name="s"),
        scratch_types=[
            pltpu.VMEM((nch, _C), jnp.int32),
            *[pltpu.VMEM((_C, d), jnp.float32) for _ in range(_NB)],
            *[pltpu.SemaphoreType.DMA for _ in range(2 * _NB)],
        ],
    )


def kernel(input, weight):
    b, s = input.shape
    d = weight.shape[-1]
    total = b * s
    idx = input.reshape(total // _C, _C).astype(jnp.int32)
    out = _make_call(total, d)(idx, weight)
    return out.reshape(b, s, d)


# speedup vs baseline: 4.5629x; 3.5088x over previous
import functools

import jax
import jax.numpy as jnp
from jax import lax
from jax.experimental import pallas as pl
from jax.experimental.pallas import tpu as pltpu
from jax.experimental.pallas import tpu_sc as plsc

_NC = 2
_NS = 16
_NW = _NC * _NS
_C = 128
_NB = 4


def _make_body(nch, nblk, d):
    def body(idx_hbm, table_hbm, out_hbm,
             idx_v, r0, r1, r2, r3, g0, g1, g2, g3, s0, s1, s2, s3):
        rows = (r0, r1, r2, r3)
        gsem = (g0, g1, g2, g3)
        ssem = (s0, s1, s2, s3)

        wid = lax.axis_index("s") * _NC + lax.axis_index("c")
        row0 = wid * nch

        pltpu.sync_copy(idx_hbm.at[pl.ds(row0, nch)], idx_v)

        def gather(c, b):
            pltpu.make_async_copy(table_hbm.at[idx_v.at[c]], rows[b], gsem[b]).start()

        def wait_gather(b):
            pltpu.make_async_copy(table_hbm.at[pl.ds(0, _C)], rows[b], gsem[b]).wait()

        def store(c, b):
            pltpu.make_async_copy(
                rows[b], out_hbm.at[pl.ds((row0 + c) * _C, _C)], ssem[b]).start()

        def wait_store(b):
            pltpu.make_async_copy(rows[b], out_hbm.at[pl.ds(0, _C)], ssem[b]).wait()

        for c in range(_NB - 1):
            gather(c, c)

        for b in range(_NB):
            pb = (b + _NB - 1) % _NB
            if b >= 1:
                wait_store(pb)
            gather(b + _NB - 1, pb)
            wait_gather(b)
            store(b, b)

        def blk(i, carry):
            c0 = i * _NB
            for b in range(_NB):
                c = c0 + b
                pb = (b + _NB - 1) % _NB
                wait_store(pb)
                gather(c + _NB - 1, pb)
                wait_gather(b)
                store(c, b)
            return carry
        lax.fori_loop(1, nblk - 1, blk, 0)

        c0 = (nblk - 1) * _NB
        wait_store(_NB - 1)
        gather(c0 + _NB - 1, _NB - 1)
        wait_gather(0)
        store(c0, 0)
        for b in range(1, _NB):
            wait_gather(b)
            store(c0 + b, b)
        for b in range(_NB):
            wait_store(b)

    return body


@functools.lru_cache(maxsize=None)
def _make_call(total, d):
    assert total % (_NW * _C) == 0
    nch = total // (_NW * _C)
    assert nch % _NB == 0 and nch // _NB >= 2
    nblk = nch // _NB

    return pl.kernel(
        _make_body(nch, nblk, d),
        out_type=jax.ShapeDtypeStruct((total, d), jnp.float32),
        mesh=plsc.VectorSubcoreMesh(core_axis_name="c", subcore_axis_name="s"),
        scratch_types=[
            pltpu.VMEM((nch, _C), jnp.int32),
            *[pltpu.VMEM((_C, d), jnp.float32) for _ in range(_NB)],
            *[pltpu.SemaphoreType.DMA for _ in range(2 * _NB)],
        ],
    )


def kernel(input, weight):
    b, s = input.shape
    d = weight.shape[-1]
    total = b * s
    idx = input.T.astype(jnp.int32).reshape(total // _C, _C)
    out = _make_call(total, d)(idx, weight)
    return out.reshape(s, b, d).transpose(1, 0, 2)


# speedup vs baseline: 4.5657x; 1.0006x over previous
import functools

import jax
import jax.numpy as jnp
from jax import lax
from jax.experimental import pallas as pl
from jax.experimental.pallas import tpu as pltpu
from jax.experimental.pallas import tpu_sc as plsc

_NC = 2
_NS = 16
_NW = _NC * _NS
_C = 256
_NB = 2


def _make_body(nch, nblk, d):
    def body(idx_hbm, table_hbm, out_hbm, idx_v, *bufs):
        rows = bufs[:_NB]
        gsem = bufs[_NB:2 * _NB]
        ssem = bufs[2 * _NB:]

        wid = lax.axis_index("s") * _NC + lax.axis_index("c")
        row0 = wid * nch

        pltpu.sync_copy(idx_hbm.at[pl.ds(row0 * _C, nch * _C)], idx_v)

        def gather(c, b):
            pltpu.make_async_copy(
                table_hbm.at[idx_v.at[pl.ds(c * _C, _C)]], rows[b], gsem[b]).start()

        def wait_gather(b):
            pltpu.make_async_copy(table_hbm.at[pl.ds(0, _C)], rows[b], gsem[b]).wait()

        def store(c, b):
            pltpu.make_async_copy(
                rows[b], out_hbm.at[pl.ds((row0 + c) * _C, _C)], ssem[b]).start()

        def wait_store(b):
            pltpu.make_async_copy(rows[b], out_hbm.at[pl.ds(0, _C)], ssem[b]).wait()

        for c in range(_NB - 1):
            gather(c, c)

        for b in range(_NB):
            pb = (b + _NB - 1) % _NB
            if b >= 1:
                wait_store(pb)
            gather(b + _NB - 1, pb)
            wait_gather(b)
            store(b, b)

        def blk(i, carry):
            c0 = i * _NB
            for b in range(_NB):
                c = c0 + b
                pb = (b + _NB - 1) % _NB
                wait_store(pb)
                gather(c + _NB - 1, pb)
                wait_gather(b)
                store(c, b)
            return carry
        lax.fori_loop(1, nblk - 1, blk, 0)

        c0 = (nblk - 1) * _NB
        wait_store(_NB - 1)
        gather(c0 + _NB - 1, _NB - 1)
        wait_gather(0)
        store(c0, 0)
        for b in range(1, _NB):
            wait_gather(b)
            store(c0 + b, b)
        for b in range(_NB):
            wait_store(b)

    return body


@functools.lru_cache(maxsize=None)
def _make_call(total, d):
    assert total % (_NW * _C) == 0
    nch = total // (_NW * _C)
    assert nch % _NB == 0 and nch // _NB >= 2
    nblk = nch // _NB

    return pl.kernel(
        _make_body(nch, nblk, d),
        out_type=jax.ShapeDtypeStruct((total, d), jnp.float32),
        mesh=plsc.VectorSubcoreMesh(core_axis_name="c", subcore_axis_name="s"),
        scratch_types=[
            pltpu.VMEM((nch * _C,), jnp.int32),
            *[pltpu.VMEM((_C, d), jnp.float32) for _ in range(_NB)],
            *[pltpu.SemaphoreType.DMA for _ in range(2 * _NB)],
        ],
    )


def kernel(input, weight):
    b, s = input.shape
    d = weight.shape[-1]
    total = b * s
    idx = input.T.astype(jnp.int32).reshape(total)
    out = _make_call(total, d)(idx, weight)
    return out.reshape(s, b, d).transpose(1, 0, 2)


# speedup vs baseline: 4.5712x; 1.0012x over previous
import functools

import jax
import jax.numpy as jnp
from jax import lax
from jax.experimental import pallas as pl
from jax.experimental.pallas import tpu as pltpu
from jax.experimental.pallas import tpu_sc as plsc

_NC = 2
_NS = 16
_NW = _NC * _NS
_C = 208
_NB = 4


def _make_body(nch, nblk, d):
    def body(idx_hbm, table_hbm, out_hbm, idx_v, *bufs):
        rows = bufs[:_NB]
        gsem = bufs[_NB:2 * _NB]
        ssem = bufs[2 * _NB:]

        wid = lax.axis_index("s") * _NC + lax.axis_index("c")
        row0 = wid * nch

        pltpu.sync_copy(idx_hbm.at[pl.ds(row0 * _C, nch * _C)], idx_v)

        def gather(c, b):
            pltpu.make_async_copy(
                table_hbm.at[idx_v.at[pl.ds(c * _C, _C)]], rows[b], gsem[b]).start()

        def wait_gather(b):
            pltpu.make_async_copy(table_hbm.at[pl.ds(0, _C)], rows[b], gsem[b]).wait()

        def store(c, b):
            pltpu.make_async_copy(
                rows[b], out_hbm.at[pl.ds((row0 + c) * _C, _C)], ssem[b]).start()

        def wait_store(b):
            pltpu.make_async_copy(rows[b], out_hbm.at[pl.ds(0, _C)], ssem[b]).wait()

        for c in range(_NB - 1):
            gather(c, c)

        for b in range(_NB):
            pb = (b + _NB - 1) % _NB
            if b >= 1:
                wait_store(pb)
            gather(b + _NB - 1, pb)
            wait_gather(b)
            store(b, b)

        def blk(i, carry):
            c0 = i * _NB
            for b in range(_NB):
                c = c0 + b
                pb = (b + _NB - 1) % _NB
                wait_store(pb)
                gather(c + _NB - 1, pb)
                wait_gather(b)
                store(c, b)
            return carry
        lax.fori_loop(1, nblk - 1, blk, 0)

        c0 = (nblk - 1) * _NB
        wait_store(_NB - 1)
        gather(c0 + _NB - 1, _NB - 1)
        wait_gather(0)
        store(c0, 0)
        for b in range(1, _NB):
            wait_gather(b)
            store(c0 + b, b)
        for b in range(_NB):
            wait_store(b)

    return body


@functools.lru_cache(maxsize=None)
def _make_call(total, d):
    assert total % (_NW * _C) == 0
    nch = total // (_NW * _C)
    assert nch % _NB == 0 and nch // _NB >= 2
    nblk = nch // _NB

    return pl.kernel(
        _make_body(nch, nblk, d),
        out_type=jax.ShapeDtypeStruct((total, d), jnp.float32),
        mesh=plsc.VectorSubcoreMesh(core_axis_name="c", subcore_axis_name="s"),
        scratch_types=[
            pltpu.VMEM((nch * _C,), jnp.int32),
            *[pltpu.VMEM((_C, d), jnp.float32) for _ in range(_NB)],
            *[pltpu.SemaphoreType.DMA for _ in range(2 * _NB)],
        ],
    )


def kernel(input, weight):
    b, s = input.shape
    d = weight.shape[-1]
    total = b * s
    idx = input.T.astype(jnp.int32).reshape(total)
    out = _make_call(total, d)(idx, weight)
    return out.reshape(s, b, d).transpose(1, 0, 2)
